# baseline (speedup 1.0000x reference)
import functools

import jax
import jax.numpy as jnp
from jax.experimental import pallas as pl
from jax.experimental.pallas import tpu as pltpu

MARGIN = 0.5
EPS = 1e-05


def _loss_block(a_ref, tcol_ref, b_ref, trow_ref, out_ref):
    a = a_ref[...]
    b = b_ref[...]
    sim = jax.lax.dot_general(
        a, b, (((1,), (1,)), ((), ())), preferred_element_type=jnp.float32
    )
    trow = trow_ref[...]
    m = sim.shape[1]
    acc = jnp.zeros((8, m), jnp.float32)
    for k in range(sim.shape[0] // 8):
        s = sim[k * 8:(k + 1) * 8, :]
        same = tcol_ref[k * 8:(k + 1) * 8, :] == trow
        t = jnp.where(same, 1.0 - s, s)
        thr = jnp.where(same, jnp.float32(EPS), jnp.float32(MARGIN))
        acc = jnp.where(t > thr, acc + t, acc)
    out_ref[...] = jnp.sum(acc).reshape(1, 1, 1)


@functools.partial(jax.jit, static_argnames=("block_m",))
def _contrastive_loss(inputs_col, targets_col, inputs_row, target_row, block_m=1024):
    n, d = inputs_col.shape
    m = inputs_row.shape[0]
    g = m // block_m
    tcol = targets_col.reshape(n, 1)
    trow = target_row.reshape(1, m)
    partials = pl.pallas_call(
        _loss_block,
        grid=(g,),
        in_specs=[
            pl.BlockSpec((n, d), lambda i: (0, 0)),
            pl.BlockSpec((n, 1), lambda i: (0, 0)),
            pl.BlockSpec((block_m, d), lambda i: (i, 0)),
            pl.BlockSpec((1, block_m), lambda i: (0, i)),
        ],
        out_specs=pl.BlockSpec((1, 1, 1), lambda i: (i, 0, 0)),
        out_shape=jax.ShapeDtypeStruct((g, 1, 1), jnp.float32),
        compiler_params=pltpu.CompilerParams(
            dimension_semantics=("parallel",),
        ),
    )(inputs_col, tcol, inputs_row, trow)
    return jnp.sum(partials) / n


def kernel(inputs_col, targets_col, inputs_row, target_row):
    return _contrastive_loss(inputs_col, targets_col, inputs_row, target_row)
